# initial kernel scaffold (unmeasured)
import jax
import jax.numpy as jnp
from jax import lax
from jax.experimental import pallas as pl
from jax.experimental.pallas import tpu as pltpu


def kernel(
    x,
):
    def body(*refs):
        pass

    out_shape = jax.ShapeDtypeStruct(..., jnp.float32)
    return pl.pallas_call(body, out_shape=out_shape)(...)


# baseline (device time: 248971 ns/iter reference)
import jax
import jax.numpy as jnp
from jax import lax
from jax.experimental import pallas as pl
from jax.experimental.pallas import tpu as pltpu

N_CHUNKS = 8


def kernel(x):
    m, n = x.shape
    rows = m // N_CHUNKS

    def body(
        x_hbm,
        out_hbm,
        xin,
        acc,
        rx,
        ry,
        in_sem,
        out_sem,
        sx_sems,
        rx_sems,
        sy_sems,
        ry_sems,
        cx_sem,
        cy_sem,
    ):
        my_x = lax.axis_index("x")
        my_y = lax.axis_index("y")
        x_nbr = (1 - my_x, my_y)
        y_nbr = (my_x, 1 - my_y)

        barrier_sem = pltpu.get_barrier_semaphore()
        for nbr in (x_nbr, y_nbr):
            pl.semaphore_signal(
                barrier_sem, inc=1,
                device_id=nbr, device_id_type=pl.DeviceIdType.MESH,
            )
        pl.semaphore_wait(barrier_sem, 2)

        def load(c):
            cp = pltpu.make_async_copy(
                x_hbm.at[pl.ds(c * rows, rows)], xin, in_sem
            )
            cp.start()
            cp.wait()
            acc[c % 2] = xin[...].astype(jnp.bfloat16)

        def x_rdma(c):
            return pltpu.make_async_remote_copy(
                src_ref=acc.at[c % 2],
                dst_ref=rx.at[c % 2],
                send_sem=sx_sems.at[c % 2],
                recv_sem=rx_sems.at[c % 2],
                device_id=x_nbr,
                device_id_type=pl.DeviceIdType.MESH,
            )

        def y_rdma(c):
            return pltpu.make_async_remote_copy(
                src_ref=acc.at[c % 2],
                dst_ref=ry.at[c % 2],
                send_sem=sy_sems.at[c % 2],
                recv_sem=ry_sems.at[c % 2],
                device_id=y_nbr,
                device_id_type=pl.DeviceIdType.MESH,
            )

        writebacks = [None] * N_CHUNKS

        load(0)
        x_rdma(0).start()
        for c in range(N_CHUNKS):
            if c + 1 < N_CHUNKS:
                load(c + 1)
                if c + 1 >= 2:
                    pl.semaphore_wait(cx_sem, 1)
                x_rdma(c + 1).start()

            x_rdma(c).wait()
            acc[c % 2] += rx[c % 2]
            pl.semaphore_signal(
                cx_sem, inc=1,
                device_id=x_nbr, device_id_type=pl.DeviceIdType.MESH,
            )

            if c >= 2:
                pl.semaphore_wait(cy_sem, 1)
            yr = y_rdma(c)
            yr.start()
            yr.wait()
            ry[c % 2] = ry[c % 2] + acc[c % 2]

            if c >= 1:
                writebacks[c - 1].wait()
                pl.semaphore_signal(
                    cy_sem, inc=1,
                    device_id=y_nbr, device_id_type=pl.DeviceIdType.MESH,
                )
            writebacks[c] = pltpu.make_async_copy(
                ry.at[c % 2], out_hbm.at[pl.ds(c * rows, rows)], out_sem
            )
            writebacks[c].start()

        writebacks[-1].wait()
        pl.semaphore_signal(
            cy_sem, inc=1,
            device_id=y_nbr, device_id_type=pl.DeviceIdType.MESH,
        )
        pl.semaphore_wait(cx_sem, 2)
        pl.semaphore_wait(cy_sem, 2)

    return pl.pallas_call(
        body,
        out_shape=jax.ShapeDtypeStruct((m, n), jnp.bfloat16),
        in_specs=[pl.BlockSpec(memory_space=pl.ANY)],
        out_specs=pl.BlockSpec(memory_space=pl.ANY),
        scratch_shapes=[
            pltpu.VMEM((rows, n), jnp.float32),
            pltpu.VMEM((2, rows, n), jnp.bfloat16),
            pltpu.VMEM((2, rows, n), jnp.bfloat16),
            pltpu.VMEM((2, rows, n), jnp.bfloat16),
            pltpu.SemaphoreType.DMA,
            pltpu.SemaphoreType.DMA,
            pltpu.SemaphoreType.DMA((2,)),
            pltpu.SemaphoreType.DMA((2,)),
            pltpu.SemaphoreType.DMA((2,)),
            pltpu.SemaphoreType.DMA((2,)),
            pltpu.SemaphoreType.REGULAR,
            pltpu.SemaphoreType.REGULAR,
        ],
        compiler_params=pltpu.CompilerParams(collective_id=0),
    )(x)


# device time: 168268 ns/iter; 1.4796x vs baseline; 1.4796x over previous
import jax
import jax.numpy as jnp
from jax import lax
from jax.experimental import pallas as pl
from jax.experimental.pallas import tpu as pltpu


def kernel(x):
    m, n = x.shape
    B = m // 4
    H = B // 2

    def body(x_hbm, out_hbm, D, st0, st1, rbufA, rbufB, rbufA2, rbufB2,
             ld_sems, wb_sems, ssems, rsems):
        i = lax.axis_index("x")
        j = lax.axis_index("y")
        x_nbr = (1 - i, j)
        y_nbr = (i, 1 - j)

        barrier_sem = pltpu.get_barrier_semaphore()
        for nbr in (x_nbr, y_nbr):
            pl.semaphore_signal(
                barrier_sem, inc=1,
                device_id=nbr, device_id_type=pl.DeviceIdType.MESH,
            )
        pl.semaphore_wait(barrier_sem, 2)

        def exchange(k, src_start, nrows, dst_ref, dst_start, nbr):
            return pltpu.make_async_remote_copy(
                src_ref=D.at[pl.ds(src_start, nrows)],
                dst_ref=dst_ref.at[pl.ds(dst_start, nrows)] if dst_start is not None else dst_ref,
                send_sem=ssems.at[k],
                recv_sem=rsems.at[k],
                device_id=nbr,
                device_id_type=pl.DeviceIdType.MESH,
            )

        def load(st, sem, block_start):
            cp = pltpu.make_async_copy(
                x_hbm.at[pl.ds(block_start, B)], st, sem
            )
            cp.start()
            return cp

        a_keep = i * B
        a_send = (1 - i) * B
        b_keep = (2 + j) * B
        b_send = (3 - j) * B
        qa = a_keep + j * H
        qa_other = a_keep + (1 - j) * H
        qb = b_keep + i * H
        qb_other = b_keep + (1 - i) * H

        l0 = load(st0, ld_sems.at[0], a_send)
        l1 = load(st1, ld_sems.at[1], b_send)
        l0.wait()
        D[pl.ds(a_send, B)] = st0[...].astype(jnp.bfloat16)
        p1x = exchange(0, a_send, B, rbufA, 0, x_nbr)
        p1x.start()
        l1.wait()
        D[pl.ds(b_send, B)] = st1[...].astype(jnp.bfloat16)
        p1y = exchange(1, b_send, B, rbufB, 0, y_nbr)
        p1y.start()

        l2 = load(st0, ld_sems.at[0], a_keep)
        l3 = load(st1, ld_sems.at[1], b_keep)
        l2.wait()
        D[pl.ds(a_keep, B)] = st0[...].astype(jnp.bfloat16)
        l3.wait()
        D[pl.ds(b_keep, B)] = st1[...].astype(jnp.bfloat16)

        p1x.wait()
        D[pl.ds(a_keep, B)] += rbufA[...]
        p2y = exchange(2, qa_other, H, rbufA2, 0, y_nbr)
        p2y.start()

        p1y.wait()
        D[pl.ds(b_keep, B)] += rbufB[...]
        p2x = exchange(3, qb_other, H, rbufB2, 0, x_nbr)
        p2x.start()

        p2y.wait()
        D[pl.ds(qa, H)] += rbufA2[...]
        p3y = exchange(4, qa, H, D, qa, y_nbr)
        p3y.start()

        p2x.wait()
        D[pl.ds(qb, H)] += rbufB2[...]
        p3x = exchange(5, qb, H, D, qb, x_nbr)
        p3x.start()

        p3y.wait()
        p3x.wait()

        p4x = exchange(6, a_keep, B, D, a_keep, x_nbr)
        p4x.start()
        p4y = exchange(7, b_keep, B, D, b_keep, y_nbr)
        p4y.start()

        w1a = pltpu.make_async_copy(
            D.at[pl.ds(a_keep, B)], out_hbm.at[pl.ds(a_keep, B)], wb_sems.at[0]
        )
        w1a.start()
        w1b = pltpu.make_async_copy(
            D.at[pl.ds(b_keep, B)], out_hbm.at[pl.ds(b_keep, B)], wb_sems.at[1]
        )
        w1b.start()

        p4x.wait()
        p4y.wait()
        w1a.wait()
        w1b.wait()

        w2a = pltpu.make_async_copy(
            D.at[pl.ds(a_send, B)], out_hbm.at[pl.ds(a_send, B)], wb_sems.at[0]
        )
        w2a.start()
        w2b = pltpu.make_async_copy(
            D.at[pl.ds(b_send, B)], out_hbm.at[pl.ds(b_send, B)], wb_sems.at[1]
        )
        w2b.start()
        w2a.wait()
        w2b.wait()

    return pl.pallas_call(
        body,
        out_shape=jax.ShapeDtypeStruct((m, n), jnp.bfloat16),
        in_specs=[pl.BlockSpec(memory_space=pl.ANY)],
        out_specs=pl.BlockSpec(memory_space=pl.ANY),
        scratch_shapes=[
            pltpu.VMEM((m, n), jnp.bfloat16),
            pltpu.VMEM((B, n), jnp.float32),
            pltpu.VMEM((B, n), jnp.float32),
            pltpu.VMEM((B, n), jnp.bfloat16),
            pltpu.VMEM((B, n), jnp.bfloat16),
            pltpu.VMEM((H, n), jnp.bfloat16),
            pltpu.VMEM((H, n), jnp.bfloat16),
            pltpu.SemaphoreType.DMA((2,)),
            pltpu.SemaphoreType.DMA((2,)),
            pltpu.SemaphoreType.DMA((8,)),
            pltpu.SemaphoreType.DMA((8,)),
        ],
        compiler_params=pltpu.CompilerParams(
            collective_id=0,
            vmem_limit_bytes=60 * 1024 * 1024,
        ),
    )(x)


# device time: 165451 ns/iter; 1.5048x vs baseline; 1.0170x over previous
import jax
import jax.numpy as jnp
from jax import lax
from jax.experimental import pallas as pl
from jax.experimental.pallas import tpu as pltpu


def kernel(x):
    m, n = x.shape
    B = m // 4
    H = B // 2

    def body(x_hbm, out_hbm, D, st0, st1, rbufA, rbufB, rbufA2, rbufB2,
             ld_sems, wb_sems, ssems, rsems):
        i = lax.axis_index("x")
        j = lax.axis_index("y")
        x_nbr = (1 - i, j)
        y_nbr = (i, 1 - j)

        barrier_sem = pltpu.get_barrier_semaphore()
        for nbr in (x_nbr, y_nbr):
            pl.semaphore_signal(
                barrier_sem, inc=1,
                device_id=nbr, device_id_type=pl.DeviceIdType.MESH,
            )
        pl.semaphore_wait(barrier_sem, 2)

        def exchange(k, src_start, nrows, dst_ref, dst_start, nbr):
            rdma = pltpu.make_async_remote_copy(
                src_ref=D.at[pl.ds(src_start, nrows)],
                dst_ref=dst_ref.at[pl.ds(dst_start, nrows)],
                send_sem=ssems.at[k],
                recv_sem=rsems.at[k],
                device_id=nbr,
                device_id_type=pl.DeviceIdType.MESH,
            )
            rdma.start()
            return rdma

        a_keep = i * B
        a_send = (1 - i) * B
        b_keep = (2 + j) * B
        b_send = (3 - j) * B
        ha_first = (1 - j) * H
        ha_second = j * H
        hb_first = (1 - i) * H
        hb_second = i * H
        qa = a_keep + j * H
        qa_other = a_keep + ha_first
        qb = b_keep + i * H
        qb_other = b_keep + hb_first

        def load(dst, sem, row_start, nrows):
            cp = pltpu.make_async_copy(
                x_hbm.at[pl.ds(row_start, nrows)], dst, sem
            )
            cp.start()
            return cp

        def cast(row_start, st_ref, st_start, nrows):
            D[pl.ds(row_start, nrows)] = (
                st_ref[pl.ds(st_start, nrows)].astype(jnp.bfloat16)
            )

        l0a = load(st0.at[pl.ds(0, H)], ld_sems.at[0], a_send + ha_first, H)
        l0b = load(st0.at[pl.ds(H, H)], ld_sems.at[1], a_send + ha_second, H)
        l1a = load(st1.at[pl.ds(0, H)], ld_sems.at[2], b_send + hb_first, H)
        l1b = load(st1.at[pl.ds(H, H)], ld_sems.at[3], b_send + hb_second, H)

        l0a.wait()
        cast(a_send + ha_first, st0, 0, H)
        p1x_h0 = exchange(0, a_send + ha_first, H, rbufA, ha_first, x_nbr)
        l0b.wait()
        cast(a_send + ha_second, st0, H, H)
        p1x_h1 = exchange(1, a_send + ha_second, H, rbufA, ha_second, x_nbr)
        l1a.wait()
        cast(b_send + hb_first, st1, 0, H)
        p1y_h0 = exchange(2, b_send + hb_first, H, rbufB, hb_first, y_nbr)
        l1b.wait()
        cast(b_send + hb_second, st1, H, H)
        p1y_h1 = exchange(3, b_send + hb_second, H, rbufB, hb_second, y_nbr)

        l2 = load(st0, ld_sems.at[0], a_keep, B)
        l3 = load(st1, ld_sems.at[1], b_keep, B)
        l2.wait()
        cast(a_keep, st0, 0, B)
        l3.wait()
        cast(b_keep, st1, 0, B)

        p1x_h0.wait()
        D[pl.ds(qa_other, H)] += rbufA[pl.ds(ha_first, H)]
        p2y = exchange(4, qa_other, H, rbufA2, 0, y_nbr)

        p1y_h0.wait()
        D[pl.ds(qb_other, H)] += rbufB[pl.ds(hb_first, H)]
        p2x = exchange(5, qb_other, H, rbufB2, 0, x_nbr)

        p1x_h1.wait()
        D[pl.ds(qa, H)] += rbufA[pl.ds(ha_second, H)]
        p1y_h1.wait()
        D[pl.ds(qb, H)] += rbufB[pl.ds(hb_second, H)]

        p2y.wait()
        D[pl.ds(qa, H)] += rbufA2[...]
        p3y = exchange(6, qa, H, D, qa, y_nbr)

        p2x.wait()
        D[pl.ds(qb, H)] += rbufB2[...]
        p3x = exchange(7, qb, H, D, qb, x_nbr)

        p3y.wait()
        p4x = exchange(8, a_keep, B, D, a_keep, x_nbr)
        w1a = pltpu.make_async_copy(
            D.at[pl.ds(a_keep, B)], out_hbm.at[pl.ds(a_keep, B)], wb_sems.at[0]
        )
        w1a.start()

        p3x.wait()
        p4y = exchange(9, b_keep, B, D, b_keep, y_nbr)
        w1b = pltpu.make_async_copy(
            D.at[pl.ds(b_keep, B)], out_hbm.at[pl.ds(b_keep, B)], wb_sems.at[1]
        )
        w1b.start()

        p4x.wait()
        w1a.wait()
        w2a = pltpu.make_async_copy(
            D.at[pl.ds(a_send, B)], out_hbm.at[pl.ds(a_send, B)], wb_sems.at[0]
        )
        w2a.start()

        p4y.wait()
        w1b.wait()
        w2b = pltpu.make_async_copy(
            D.at[pl.ds(b_send, B)], out_hbm.at[pl.ds(b_send, B)], wb_sems.at[1]
        )
        w2b.start()
        w2a.wait()
        w2b.wait()

    return pl.pallas_call(
        body,
        out_shape=jax.ShapeDtypeStruct((m, n), jnp.bfloat16),
        in_specs=[pl.BlockSpec(memory_space=pl.ANY)],
        out_specs=pl.BlockSpec(memory_space=pl.ANY),
        scratch_shapes=[
            pltpu.VMEM((m, n), jnp.bfloat16),
            pltpu.VMEM((B, n), jnp.float32),
            pltpu.VMEM((B, n), jnp.float32),
            pltpu.VMEM((B, n), jnp.bfloat16),
            pltpu.VMEM((B, n), jnp.bfloat16),
            pltpu.VMEM((H, n), jnp.bfloat16),
            pltpu.VMEM((H, n), jnp.bfloat16),
            pltpu.SemaphoreType.DMA((4,)),
            pltpu.SemaphoreType.DMA((2,)),
            pltpu.SemaphoreType.DMA((10,)),
            pltpu.SemaphoreType.DMA((10,)),
        ],
        compiler_params=pltpu.CompilerParams(
            collective_id=0,
            vmem_limit_bytes=60 * 1024 * 1024,
        ),
    )(x)


# device time: 163446 ns/iter; 1.5233x vs baseline; 1.0123x over previous
import jax
import jax.numpy as jnp
from jax import lax
from jax.experimental import pallas as pl
from jax.experimental.pallas import tpu as pltpu


def kernel(x):
    m, n = x.shape
    B = m // 4
    H = B // 2

    def body(x_hbm, out_hbm, D, st0, st1, rbufA, rbufB, rbufA2, rbufB2,
             ld_sems, wb_sems, ssems, rsems):
        i = lax.axis_index("x")
        j = lax.axis_index("y")
        x_nbr = (1 - i, j)
        y_nbr = (i, 1 - j)

        barrier_sem = pltpu.get_barrier_semaphore()
        for nbr in (x_nbr, y_nbr):
            pl.semaphore_signal(
                barrier_sem, inc=1,
                device_id=nbr, device_id_type=pl.DeviceIdType.MESH,
            )
        pl.semaphore_wait(barrier_sem, 2)

        def exchange(k, src_start, nrows, dst_ref, dst_start, nbr):
            rdma = pltpu.make_async_remote_copy(
                src_ref=D.at[pl.ds(src_start, nrows)],
                dst_ref=dst_ref.at[pl.ds(dst_start, nrows)],
                send_sem=ssems.at[k],
                recv_sem=rsems.at[k],
                device_id=nbr,
                device_id_type=pl.DeviceIdType.MESH,
            )
            rdma.start()
            return rdma

        a_keep = i * B
        a_send = (1 - i) * B
        b_keep = (2 + j) * B
        b_send = (3 - j) * B
        ha_first = (1 - j) * H
        ha_second = j * H
        hb_first = (1 - i) * H
        hb_second = i * H
        qa = a_keep + j * H
        qa_other = a_keep + ha_first
        qb = b_keep + i * H
        qb_other = b_keep + hb_first

        def load(dst, sem, row_start, nrows):
            cp = pltpu.make_async_copy(
                x_hbm.at[pl.ds(row_start, nrows)], dst, sem
            )
            cp.start()
            return cp

        def cast(row_start, st_ref, st_start, nrows):
            D[pl.ds(row_start, nrows)] = (
                st_ref[pl.ds(st_start, nrows)].astype(jnp.bfloat16)
            )

        l0a = load(st0.at[pl.ds(0, H)], ld_sems.at[0], a_send + ha_first, H)
        l1a = load(st1.at[pl.ds(0, H)], ld_sems.at[2], b_send + hb_first, H)

        l0a.wait()
        cast(a_send + ha_first, st0, 0, H)
        p1x_h0 = exchange(0, a_send + ha_first, H, rbufA, ha_first, x_nbr)
        l0b = load(st0.at[pl.ds(H, H)], ld_sems.at[1], a_send + ha_second, H)
        l1a.wait()
        cast(b_send + hb_first, st1, 0, H)
        p1y_h0 = exchange(2, b_send + hb_first, H, rbufB, hb_first, y_nbr)
        l1b = load(st1.at[pl.ds(H, H)], ld_sems.at[3], b_send + hb_second, H)

        l0b.wait()
        cast(a_send + ha_second, st0, H, H)
        p1x_h1 = exchange(1, a_send + ha_second, H, rbufA, ha_second, x_nbr)
        l1b.wait()
        cast(b_send + hb_second, st1, H, H)
        p1y_h1 = exchange(3, b_send + hb_second, H, rbufB, hb_second, y_nbr)

        l2 = load(st0, ld_sems.at[0], a_keep, B)
        l3 = load(st1, ld_sems.at[1], b_keep, B)
        l2.wait()
        cast(a_keep, st0, 0, B)
        l3.wait()
        cast(b_keep, st1, 0, B)

        p1x_h0.wait()
        D[pl.ds(qa_other, H)] += rbufA[pl.ds(ha_first, H)]
        p2y = exchange(4, qa_other, H, rbufA2, 0, y_nbr)

        p1y_h0.wait()
        D[pl.ds(qb_other, H)] += rbufB[pl.ds(hb_first, H)]
        p2x = exchange(5, qb_other, H, rbufB2, 0, x_nbr)

        p1x_h1.wait()
        D[pl.ds(qa, H)] += rbufA[pl.ds(ha_second, H)]
        p1y_h1.wait()
        D[pl.ds(qb, H)] += rbufB[pl.ds(hb_second, H)]

        p2y.wait()
        D[pl.ds(qa, H)] += rbufA2[...]
        p3y = exchange(6, qa, H, D, qa, y_nbr)

        p2x.wait()
        D[pl.ds(qb, H)] += rbufB2[...]
        p3x = exchange(7, qb, H, D, qb, x_nbr)

        def writeback(row_start, nrows, sem):
            cp = pltpu.make_async_copy(
                D.at[pl.ds(row_start, nrows)],
                out_hbm.at[pl.ds(row_start, nrows)],
                sem,
            )
            cp.start()
            return cp

        p3y.wait()
        p4x_h0 = exchange(8, a_keep, H, D, a_keep, x_nbr)
        p4x_h1 = exchange(9, a_keep + H, H, D, a_keep + H, x_nbr)
        w1a = writeback(a_keep, B, wb_sems.at[0])

        p3x.wait()
        p4y_h0 = exchange(10, b_keep, H, D, b_keep, y_nbr)
        p4y_h1 = exchange(11, b_keep + H, H, D, b_keep + H, y_nbr)
        w1b = writeback(b_keep, B, wb_sems.at[1])

        p4x_h0.wait()
        w2a0 = writeback(a_send, H, wb_sems.at[2])
        p4x_h1.wait()
        w2a1 = writeback(a_send + H, H, wb_sems.at[3])
        p4y_h0.wait()
        w2b0 = writeback(b_send, H, wb_sems.at[4])
        p4y_h1.wait()
        w2b1 = writeback(b_send + H, H, wb_sems.at[5])

        for cp in (w1a, w1b, w2a0, w2a1, w2b0, w2b1):
            cp.wait()

    return pl.pallas_call(
        body,
        out_shape=jax.ShapeDtypeStruct((m, n), jnp.bfloat16),
        in_specs=[pl.BlockSpec(memory_space=pl.ANY)],
        out_specs=pl.BlockSpec(memory_space=pl.ANY),
        scratch_shapes=[
            pltpu.VMEM((m, n), jnp.bfloat16),
            pltpu.VMEM((B, n), jnp.float32),
            pltpu.VMEM((B, n), jnp.float32),
            pltpu.VMEM((B, n), jnp.bfloat16),
            pltpu.VMEM((B, n), jnp.bfloat16),
            pltpu.VMEM((H, n), jnp.bfloat16),
            pltpu.VMEM((H, n), jnp.bfloat16),
            pltpu.SemaphoreType.DMA((4,)),
            pltpu.SemaphoreType.DMA((6,)),
            pltpu.SemaphoreType.DMA((12,)),
            pltpu.SemaphoreType.DMA((12,)),
        ],
        compiler_params=pltpu.CompilerParams(
            collective_id=0,
            vmem_limit_bytes=60 * 1024 * 1024,
        ),
    )(x)


# device time: 157463 ns/iter; 1.5811x vs baseline; 1.0380x over previous
import jax
import jax.numpy as jnp
from jax import lax
from jax.experimental import pallas as pl
from jax.experimental.pallas import tpu as pltpu


def kernel(x):
    m, n = x.shape
    B = m // 4
    H = B // 2
    Q = H // 2

    def body(x_hbm, out_hbm, D, st0, st1, rbufA, rbufB, rbufA2, rbufB2,
             ld_sems, wb_sems, ssems, rsems):
        i = lax.axis_index("x")
        j = lax.axis_index("y")
        x_nbr = (1 - i, j)
        y_nbr = (i, 1 - j)

        barrier_sem = pltpu.get_barrier_semaphore()
        for nbr in (x_nbr, y_nbr):
            pl.semaphore_signal(
                barrier_sem, inc=1,
                device_id=nbr, device_id_type=pl.DeviceIdType.MESH,
            )
        pl.semaphore_wait(barrier_sem, 2)

        def exchange(k, src_start, nrows, dst_ref, dst_start, nbr):
            rdma = pltpu.make_async_remote_copy(
                src_ref=D.at[pl.ds(src_start, nrows)],
                dst_ref=dst_ref.at[pl.ds(dst_start, nrows)],
                send_sem=ssems.at[k],
                recv_sem=rsems.at[k],
                device_id=nbr,
                device_id_type=pl.DeviceIdType.MESH,
            )
            rdma.start()
            return rdma

        a_keep = i * B
        a_send = (1 - i) * B
        b_keep = (2 + j) * B
        b_send = (3 - j) * B
        ha_first = (1 - j) * H
        ha_second = j * H
        hb_first = (1 - i) * H
        hb_second = i * H
        qa = a_keep + ha_second
        qa_other = a_keep + ha_first
        qb = b_keep + hb_second
        qb_other = b_keep + hb_first

        def load(sem_k, st_ref, st_start, row_start, nrows):
            cp = pltpu.make_async_copy(
                x_hbm.at[pl.ds(row_start, nrows)],
                st_ref.at[pl.ds(st_start, nrows)],
                ld_sems.at[sem_k],
            )
            cp.start()
            return cp

        def cast(row_start, st_ref, st_start, nrows):
            D[pl.ds(row_start, nrows)] = (
                st_ref[pl.ds(st_start, nrows)].astype(jnp.bfloat16)
            )

        def writeback(sem_k, row_start, nrows):
            cp = pltpu.make_async_copy(
                D.at[pl.ds(row_start, nrows)],
                out_hbm.at[pl.ds(row_start, nrows)],
                wb_sems.at[sem_k],
            )
            cp.start()
            return cp

        l_ax0 = load(0, st0, 0, a_send + ha_first, Q)
        l_bx0 = load(4, st1, 0, b_send + hb_first, Q)

        l_ax0.wait()
        cast(a_send + ha_first, st0, 0, Q)
        p1x_q0 = exchange(0, a_send + ha_first, Q, rbufA, ha_first, x_nbr)
        l_ax1 = load(1, st0, Q, a_send + ha_first + Q, Q)

        l_bx0.wait()
        cast(b_send + hb_first, st1, 0, Q)
        p1y_q0 = exchange(9, b_send + hb_first, Q, rbufB, hb_first, y_nbr)
        l_bx1 = load(5, st1, Q, b_send + hb_first + Q, Q)

        l_ax1.wait()
        cast(a_send + ha_first + Q, st0, Q, Q)
        p1x_q1 = exchange(
            1, a_send + ha_first + Q, Q, rbufA, ha_first + Q, x_nbr
        )
        l_ah1 = load(2, st0, H, a_send + ha_second, H)

        l_bx1.wait()
        cast(b_send + hb_first + Q, st1, Q, Q)
        p1y_q1 = exchange(
            10, b_send + hb_first + Q, Q, rbufB, hb_first + Q, y_nbr
        )
        l_bh1 = load(6, st1, H, b_send + hb_second, H)

        l_ah1.wait()
        cast(a_send + ha_second, st0, H, H)
        p1x_h1 = exchange(2, a_send + ha_second, H, rbufA, ha_second, x_nbr)
        l_bh1.wait()
        cast(b_send + hb_second, st1, H, H)
        p1y_h1 = exchange(11, b_send + hb_second, H, rbufB, hb_second, y_nbr)

        l_ak = load(3, st0, 0, a_keep, B)
        l_bk = load(7, st1, 0, b_keep, B)
        l_ak.wait()
        cast(a_keep, st0, 0, B)
        l_bk.wait()
        cast(b_keep, st1, 0, B)

        p1x_q0.wait()
        D[pl.ds(qa_other, Q)] += rbufA[pl.ds(ha_first, Q)]
        p2y_q0 = exchange(12, qa_other, Q, rbufA2, 0, y_nbr)

        p1y_q0.wait()
        D[pl.ds(qb_other, Q)] += rbufB[pl.ds(hb_first, Q)]
        p2x_q0 = exchange(3, qb_other, Q, rbufB2, 0, x_nbr)

        p1x_q1.wait()
        D[pl.ds(qa_other + Q, Q)] += rbufA[pl.ds(ha_first + Q, Q)]
        p2y_q1 = exchange(13, qa_other + Q, Q, rbufA2, Q, y_nbr)

        p1y_q1.wait()
        D[pl.ds(qb_other + Q, Q)] += rbufB[pl.ds(hb_first + Q, Q)]
        p2x_q1 = exchange(4, qb_other + Q, Q, rbufB2, Q, x_nbr)

        p1x_h1.wait()
        D[pl.ds(qa, H)] += rbufA[pl.ds(ha_second, H)]
        p1y_h1.wait()
        D[pl.ds(qb, H)] += rbufB[pl.ds(hb_second, H)]

        p2y_q0.wait()
        D[pl.ds(qa, Q)] += rbufA2[pl.ds(0, Q)]
        p3y_q0 = exchange(14, qa, Q, D, qa, y_nbr)

        p2x_q0.wait()
        D[pl.ds(qb, Q)] += rbufB2[pl.ds(0, Q)]
        p3x_q0 = exchange(5, qb, Q, D, qb, x_nbr)

        p2y_q1.wait()
        D[pl.ds(qa + Q, Q)] += rbufA2[pl.ds(Q, Q)]
        p3y_q1 = exchange(15, qa + Q, Q, D, qa + Q, y_nbr)

        p2x_q1.wait()
        D[pl.ds(qb + Q, Q)] += rbufB2[pl.ds(Q, Q)]
        p3x_q1 = exchange(6, qb + Q, Q, D, qb + Q, x_nbr)

        p4x_qa = exchange(7, qa, H, D, qa, x_nbr)
        p4y_qb = exchange(16, qb, H, D, qb, y_nbr)

        p3y_q0.wait()
        p3y_q1.wait()
        p4x_qao = exchange(8, qa_other, H, D, qa_other, x_nbr)
        w1a = writeback(0, a_keep, B)

        p3x_q0.wait()
        p3x_q1.wait()
        p4y_qbo = exchange(17, qb_other, H, D, qb_other, y_nbr)
        w1b = writeback(1, b_keep, B)

        p4x_qa.wait()
        w2a0 = writeback(2, a_send + ha_second, H)
        p4y_qb.wait()
        w2b0 = writeback(3, b_send + hb_second, H)
        p4x_qao.wait()
        w2a1 = writeback(4, a_send + ha_first, H)
        p4y_qbo.wait()
        w2b1 = writeback(5, b_send + hb_first, H)

        for cp in (w1a, w1b, w2a0, w2b0, w2a1, w2b1):
            cp.wait()

    return pl.pallas_call(
        body,
        out_shape=jax.ShapeDtypeStruct((m, n), jnp.bfloat16),
        in_specs=[pl.BlockSpec(memory_space=pl.ANY)],
        out_specs=pl.BlockSpec(memory_space=pl.ANY),
        scratch_shapes=[
            pltpu.VMEM((m, n), jnp.bfloat16),
            pltpu.VMEM((B, n), jnp.float32),
            pltpu.VMEM((B, n), jnp.float32),
            pltpu.VMEM((B, n), jnp.bfloat16),
            pltpu.VMEM((B, n), jnp.bfloat16),
            pltpu.VMEM((H, n), jnp.bfloat16),
            pltpu.VMEM((H, n), jnp.bfloat16),
            pltpu.SemaphoreType.DMA((8,)),
            pltpu.SemaphoreType.DMA((6,)),
            pltpu.SemaphoreType.DMA((18,)),
            pltpu.SemaphoreType.DMA((18,)),
        ],
        compiler_params=pltpu.CompilerParams(
            collective_id=0,
            vmem_limit_bytes=60 * 1024 * 1024,
        ),
    )(x)


# device time: 155729 ns/iter; 1.5987x vs baseline; 1.0111x over previous
import jax
import jax.numpy as jnp
from jax import lax
from jax.experimental import pallas as pl
from jax.experimental.pallas import tpu as pltpu


def kernel(x):
    m, n = x.shape
    B = m // 4
    H = B // 2
    Q = H // 2

    def body(x_hbm, out_hbm, D, st0, st1, rbufA, rbufB, rbufA2, rbufB2,
             ld_sems, wb_sems, ssems, rsems):
        i = lax.axis_index("x")
        j = lax.axis_index("y")
        x_nbr = (1 - i, j)
        y_nbr = (i, 1 - j)

        def exchange(k, src_start, nrows, dst_ref, dst_start, nbr):
            rdma = pltpu.make_async_remote_copy(
                src_ref=D.at[pl.ds(src_start, nrows)],
                dst_ref=dst_ref.at[pl.ds(dst_start, nrows)],
                send_sem=ssems.at[k],
                recv_sem=rsems.at[k],
                device_id=nbr,
                device_id_type=pl.DeviceIdType.MESH,
            )
            rdma.start()
            return rdma

        a_keep = i * B
        a_send = (1 - i) * B
        b_keep = (2 + j) * B
        b_send = (3 - j) * B
        ha_first = (1 - j) * H
        ha_second = j * H
        hb_first = (1 - i) * H
        hb_second = i * H
        qa = a_keep + ha_second
        qa_other = a_keep + ha_first
        qb = b_keep + hb_second
        qb_other = b_keep + hb_first

        def load(sem_k, st_ref, st_start, row_start, nrows):
            cp = pltpu.make_async_copy(
                x_hbm.at[pl.ds(row_start, nrows)],
                st_ref.at[pl.ds(st_start, nrows)],
                ld_sems.at[sem_k],
            )
            cp.start()
            return cp

        def cast(row_start, st_ref, st_start, nrows):
            D[pl.ds(row_start, nrows)] = (
                st_ref[pl.ds(st_start, nrows)].astype(jnp.bfloat16)
            )

        def writeback(sem_k, row_start, nrows):
            cp = pltpu.make_async_copy(
                D.at[pl.ds(row_start, nrows)],
                out_hbm.at[pl.ds(row_start, nrows)],
                wb_sems.at[sem_k],
            )
            cp.start()
            return cp

        l_ax0 = load(0, st0, 0, a_send + ha_first, Q)
        l_bx0 = load(4, st1, 0, b_send + hb_first, Q)

        barrier_sem = pltpu.get_barrier_semaphore()
        for nbr in (x_nbr, y_nbr):
            pl.semaphore_signal(
                barrier_sem, inc=1,
                device_id=nbr, device_id_type=pl.DeviceIdType.MESH,
            )
        pl.semaphore_wait(barrier_sem, 2)

        l_ax0.wait()
        cast(a_send + ha_first, st0, 0, Q)
        p1x_q0 = exchange(0, a_send + ha_first, Q, rbufA, ha_first, x_nbr)
        l_ax1 = load(1, st0, Q, a_send + ha_first + Q, Q)

        l_bx0.wait()
        cast(b_send + hb_first, st1, 0, Q)
        p1y_q0 = exchange(9, b_send + hb_first, Q, rbufB, hb_first, y_nbr)
        l_bx1 = load(5, st1, Q, b_send + hb_first + Q, Q)

        l_ax1.wait()
        cast(a_send + ha_first + Q, st0, Q, Q)
        p1x_q1 = exchange(
            1, a_send + ha_first + Q, Q, rbufA, ha_first + Q, x_nbr
        )
        l_ah1 = load(2, st0, H, a_send + ha_second, H)

        l_bx1.wait()
        cast(b_send + hb_first + Q, st1, Q, Q)
        p1y_q1 = exchange(
            10, b_send + hb_first + Q, Q, rbufB, hb_first + Q, y_nbr
        )
        l_bh1 = load(6, st1, H, b_send + hb_second, H)

        l_ah1.wait()
        cast(a_send + ha_second, st0, H, H)
        p1x_h1 = exchange(2, a_send + ha_second, H, rbufA, ha_second, x_nbr)
        l_bh1.wait()
        cast(b_send + hb_second, st1, H, H)
        p1y_h1 = exchange(11, b_send + hb_second, H, rbufB, hb_second, y_nbr)

        l_ak = load(3, st0, 0, a_keep, B)
        l_bk = load(7, st1, 0, b_keep, B)
        l_ak.wait()
        cast(a_keep, st0, 0, B)
        l_bk.wait()
        cast(b_keep, st1, 0, B)

        p1x_q0.wait()
        D[pl.ds(qa_other, Q)] += rbufA[pl.ds(ha_first, Q)]
        p2y_q0 = exchange(12, qa_other, Q, rbufA2, 0, y_nbr)

        p1y_q0.wait()
        D[pl.ds(qb_other, Q)] += rbufB[pl.ds(hb_first, Q)]
        p2x_q0 = exchange(3, qb_other, Q, rbufB2, 0, x_nbr)

        p1x_q1.wait()
        D[pl.ds(qa_other + Q, Q)] += rbufA[pl.ds(ha_first + Q, Q)]
        p2y_q1 = exchange(13, qa_other + Q, Q, rbufA2, Q, y_nbr)

        p1y_q1.wait()
        D[pl.ds(qb_other + Q, Q)] += rbufB[pl.ds(hb_first + Q, Q)]
        p2x_q1 = exchange(4, qb_other + Q, Q, rbufB2, Q, x_nbr)

        p1x_h1.wait()
        D[pl.ds(qa, H)] += rbufA[pl.ds(ha_second, H)]
        p1y_h1.wait()
        D[pl.ds(qb, H)] += rbufB[pl.ds(hb_second, H)]

        p2y_q0.wait()
        D[pl.ds(qa, Q)] += rbufA2[pl.ds(0, Q)]
        p3y_q0 = exchange(14, qa, Q, D, qa, y_nbr)

        p2x_q0.wait()
        D[pl.ds(qb, Q)] += rbufB2[pl.ds(0, Q)]
        p3x_q0 = exchange(5, qb, Q, D, qb, x_nbr)

        p2y_q1.wait()
        D[pl.ds(qa + Q, Q)] += rbufA2[pl.ds(Q, Q)]
        p3y_q1 = exchange(15, qa + Q, Q, D, qa + Q, y_nbr)

        p2x_q1.wait()
        D[pl.ds(qb + Q, Q)] += rbufB2[pl.ds(Q, Q)]
        p3x_q1 = exchange(6, qb + Q, Q, D, qb + Q, x_nbr)

        p4x_qa = exchange(7, qa, H, D, qa, x_nbr)
        p4y_qb = exchange(16, qb, H, D, qb, y_nbr)

        p3y_q0.wait()
        p3y_q1.wait()
        p4x_qao0 = exchange(8, qa_other, Q, D, qa_other, x_nbr)
        p4x_qao1 = exchange(18, qa_other + Q, Q, D, qa_other + Q, x_nbr)
        w1a = writeback(0, a_keep, B)

        p3x_q0.wait()
        p3x_q1.wait()
        p4y_qbo0 = exchange(17, qb_other, Q, D, qb_other, y_nbr)
        p4y_qbo1 = exchange(19, qb_other + Q, Q, D, qb_other + Q, y_nbr)
        w1b = writeback(1, b_keep, B)

        p4x_qa.wait()
        w2a0 = writeback(2, a_send + ha_second, H)
        p4y_qb.wait()
        w2b0 = writeback(3, b_send + hb_second, H)
        p4x_qao0.wait()
        w2a1 = writeback(4, a_send + ha_first, Q)
        p4y_qbo0.wait()
        w2b1 = writeback(5, b_send + hb_first, Q)
        p4x_qao1.wait()
        w2a2 = writeback(6, a_send + ha_first + Q, Q)
        p4y_qbo1.wait()
        w2b2 = writeback(7, b_send + hb_first + Q, Q)

        for cp in (w1a, w1b, w2a0, w2b0, w2a1, w2b1, w2a2, w2b2):
            cp.wait()

    return pl.pallas_call(
        body,
        out_shape=jax.ShapeDtypeStruct((m, n), jnp.bfloat16),
        in_specs=[pl.BlockSpec(memory_space=pl.ANY)],
        out_specs=pl.BlockSpec(memory_space=pl.ANY),
        scratch_shapes=[
            pltpu.VMEM((m, n), jnp.bfloat16),
            pltpu.VMEM((B, n), jnp.float32),
            pltpu.VMEM((B, n), jnp.float32),
            pltpu.VMEM((B, n), jnp.bfloat16),
            pltpu.VMEM((B, n), jnp.bfloat16),
            pltpu.VMEM((H, n), jnp.bfloat16),
            pltpu.VMEM((H, n), jnp.bfloat16),
            pltpu.SemaphoreType.DMA((8,)),
            pltpu.SemaphoreType.DMA((8,)),
            pltpu.SemaphoreType.DMA((20,)),
            pltpu.SemaphoreType.DMA((20,)),
        ],
        compiler_params=pltpu.CompilerParams(
            collective_id=0,
            vmem_limit_bytes=60 * 1024 * 1024,
        ),
    )(x)
